# Initial kernel scaffold; baseline (speedup 1.0000x reference)
import functools

import jax
import jax.numpy as jnp
from jax import lax
from jax.experimental import pallas as pl
from jax.experimental.pallas import tpu as pltpu
from jax.experimental.pallas import tpu_sc as plsc

EMBED_DIM = 32
BATCH = 16384
HIST = 200
B_TOTAL = BATCH * HIST
IW = 128
NW = 32
ROWS_PER_W = B_TOTAL // (NW * IW)
K = 16
N_CHUNK = ROWS_PER_W // K
CHUNK = K * IW


def _make_gather():
    mesh = plsc.VectorSubcoreMesh(core_axis_name="c", subcore_axis_name="s")

    @functools.partial(
        pl.kernel,
        mesh=mesh,
        out_type=jax.ShapeDtypeStruct((B_TOTAL, EMBED_DIM), jnp.float32),
        scratch_types=[
            pltpu.VMEM((K, IW), jnp.int32),
            pltpu.VMEM((CHUNK, EMBED_DIM), jnp.float32),
            pltpu.SemaphoreType.DMA,
        ],
    )
    def gather_kernel(idx_hbm, table_hbm, out_hbm, idx_v, rows_v, sem):
        wid = lax.axis_index("s") * 2 + lax.axis_index("c")
        row_base = wid * ROWS_PER_W

        def body(ci, carry):
            row_off = row_base + ci * K
            pltpu.sync_copy(idx_hbm.at[pl.ds(row_off, K)], idx_v)
            copies = []
            for j in range(K):
                copies.append(
                    pltpu.async_copy(
                        table_hbm.at[idx_v.at[j]],
                        rows_v.at[pl.ds(j * IW, IW)],
                        sem,
                    )
                )
            for c in copies:
                c.wait()
            pltpu.sync_copy(rows_v, out_hbm.at[pl.ds(row_off * IW, CHUNK)])
            return carry

        lax.fori_loop(0, N_CHUNK, body, 0)

    return gather_kernel


_gather = _make_gather()


def kernel(token_ids, embed_weight):
    idx = token_ids.astype(jnp.int32).reshape(B_TOTAL // IW, IW)
    out = _gather(idx, embed_weight)
    return out.reshape(BATCH, HIST, EMBED_DIM)


# speedup vs baseline: 4.9471x; 4.9471x over previous
import functools

import jax
import jax.numpy as jnp
from jax import lax
from jax.experimental import pallas as pl
from jax.experimental.pallas import tpu as pltpu
from jax.experimental.pallas import tpu_sc as plsc

EMBED_DIM = 32
BATCH = 16384
HIST = 200
B_TOTAL = BATCH * HIST
IW = 128
NW = 32
ROWS_PER_W = B_TOTAL // (NW * IW)
K = 16
N_CHUNK = ROWS_PER_W // K
CHUNK = K * IW


def _make_gather():
    mesh = plsc.VectorSubcoreMesh(core_axis_name="c", subcore_axis_name="s")

    @functools.partial(
        pl.kernel,
        mesh=mesh,
        compiler_params=pltpu.CompilerParams(use_tc_tiling_on_sc=False),
        out_type=jax.ShapeDtypeStruct((B_TOTAL, EMBED_DIM), jnp.float32),
        scratch_types=[
            pltpu.VMEM((K, IW), jnp.int32),
            pltpu.VMEM((CHUNK, EMBED_DIM), jnp.float32),
            pltpu.SemaphoreType.DMA,
        ],
    )
    def gather_kernel(idx_hbm, table_hbm, out_hbm, idx_v, rows_v, sem):
        wid = lax.axis_index("s") * 2 + lax.axis_index("c")
        row_base = wid * ROWS_PER_W

        def body(ci, carry):
            row_off = row_base + ci * K
            pltpu.sync_copy(idx_hbm.at[pl.ds(row_off, K)], idx_v)
            copies = []
            for j in range(K):
                copies.append(
                    pltpu.async_copy(
                        table_hbm.at[idx_v.at[j]],
                        rows_v.at[pl.ds(j * IW, IW)],
                        sem,
                    )
                )
            for c in copies:
                c.wait()
            pltpu.sync_copy(rows_v, out_hbm.at[pl.ds(row_off * IW, CHUNK)])
            return carry

        lax.fori_loop(0, N_CHUNK, body, 0)

    return gather_kernel


_gather = _make_gather()


def kernel(token_ids, embed_weight):
    idx = token_ids.astype(jnp.int32).reshape(B_TOTAL // IW, IW)
    out = _gather(idx, embed_weight)
    return out.reshape(BATCH, HIST, EMBED_DIM)


# speedup vs baseline: 4.9690x; 1.0044x over previous
import functools

import jax
import jax.numpy as jnp
from jax import lax
from jax.experimental import pallas as pl
from jax.experimental.pallas import tpu as pltpu
from jax.experimental.pallas import tpu_sc as plsc

EMBED_DIM = 32
BATCH = 16384
HIST = 200
B_TOTAL = BATCH * HIST
IW = 128
NW = 32
ROWS_PER_W = B_TOTAL // (NW * IW)
K = 10
N_CHUNK = ROWS_PER_W // K
CHUNK = K * IW
NBUF = 2
N_STEP = N_CHUNK // NBUF


def _make_gather():
    mesh = plsc.VectorSubcoreMesh(core_axis_name="c", subcore_axis_name="s")

    @functools.partial(
        pl.kernel,
        mesh=mesh,
        compiler_params=pltpu.CompilerParams(use_tc_tiling_on_sc=False),
        out_type=jax.ShapeDtypeStruct((B_TOTAL, EMBED_DIM), jnp.float32),
        scratch_types=[
            pltpu.VMEM((K, IW), jnp.int32),
            pltpu.VMEM((K, IW), jnp.int32),
            pltpu.VMEM((CHUNK, EMBED_DIM), jnp.float32),
            pltpu.VMEM((CHUNK, EMBED_DIM), jnp.float32),
            pltpu.SemaphoreType.DMA,
            pltpu.SemaphoreType.DMA,
            pltpu.SemaphoreType.DMA,
            pltpu.SemaphoreType.DMA,
        ],
    )
    def gather_kernel(idx_hbm, table_hbm, out_hbm,
                      idx0, idx1, rows0, rows1, sg0, sg1, ss0, ss1):
        idxs = (idx0, idx1)
        rows = (rows0, rows1)
        sgs = (sg0, sg1)
        sss = (ss0, ss1)
        wid = lax.axis_index("s") * 2 + lax.axis_index("c")
        row_base = wid * ROWS_PER_W

        def load_and_fire(c, b):
            row_off = row_base + c * K
            pltpu.sync_copy(idx_hbm.at[pl.ds(row_off, K)], idxs[b])
            for j in range(K):
                pltpu.async_copy(
                    table_hbm.at[idxs[b].at[j]],
                    rows[b].at[pl.ds(j * IW, IW)],
                    sgs[b],
                )

        def drain_gather(b):
            pltpu.make_async_copy(
                table_hbm.at[idxs[b].at[0]], rows[b].at[pl.ds(0, CHUNK)], sgs[b]
            ).wait()

        def fire_store(c, b):
            row_off = row_base + c * K
            pltpu.async_copy(
                rows[b], out_hbm.at[pl.ds(row_off * IW, CHUNK)], sss[b]
            )

        def drain_store(b):
            pltpu.make_async_copy(
                rows[b], out_hbm.at[pl.ds(row_base * IW, CHUNK)], sss[b]
            ).wait()

        for b in range(NBUF):
            load_and_fire(b, b)

        def body(ci, carry):
            for b in range(NBUF):
                c = ci * NBUF + b
                drain_gather(b)
                fire_store(c, b)

                @pl.when(ci < N_STEP - 1)
                def _():
                    drain_store(b)
                    load_and_fire(c + NBUF, b)

            return carry

        lax.fori_loop(0, N_STEP, body, 0)
        for b in range(NBUF):
            drain_store(b)

    return gather_kernel


_gather = _make_gather()


def kernel(token_ids, embed_weight):
    idx = token_ids.astype(jnp.int32).reshape(B_TOTAL // IW, IW)
    out = _gather(idx, embed_weight)
    return out.reshape(BATCH, HIST, EMBED_DIM)
